# Initial kernel scaffold; baseline (speedup 1.0000x reference)
import jax
import jax.numpy as jnp
from jax.experimental import pallas as pl


def kernel(x, edge_index, batch, W1, b1, W2, b2, W3, b3, W4, b4, Wf, bf):
    raise NotImplementedError("write your pallas kernel here")


# speedup vs baseline: 16.1265x; 16.1265x over previous
import functools

import jax
import jax.numpy as jnp
from jax import lax
from jax.experimental import pallas as pl
from jax.experimental.pallas import tpu as pltpu
from jax.experimental.pallas import tpu_sc as plsc

N_NODES = 10000
N_PAD = 10240
N_EDGES = 320000
N_BATCHES = N_EDGES // 128
D_OUT = 32
N_GRAPHS = 64

_MESH = plsc.VectorSubcoreMesh(core_axis_name="c", subcore_axis_name="s")
_SC_PARAMS = pltpu.CompilerParams(use_tc_tiling_on_sc=False)


def _prep_body(x_ref, w1, w2, w3, w4, wf, b1, b2, b3, b4, z0_ref, cs_ref):
    f32 = jnp.float32
    wc4 = jnp.dot(w4[...], wf[...], preferred_element_type=f32)
    wc3 = jnp.dot(w3[...], wc4, preferred_element_type=f32)
    wc2 = jnp.dot(w2[...], wc3, preferred_element_type=f32)
    wc1 = jnp.dot(w1[...], wc2, preferred_element_type=f32)
    z0_ref[...] = jnp.dot(x_ref[...], wc1, preferred_element_type=f32)
    cs_ref[0:1, :] = jnp.dot(b1[...], wc2, preferred_element_type=f32)
    cs_ref[1:2, :] = jnp.dot(b2[...], wc3, preferred_element_type=f32)
    cs_ref[2:3, :] = jnp.dot(b3[...], wc4, preferred_element_type=f32)
    cs_ref[3:4, :] = jnp.dot(b4[...], wf[...], preferred_element_type=f32)


def _scale_body(z_ref, deg_ref, y_ref, dinv_ref, dinv2_ref):
    deg = deg_ref[...] + 1.0
    dinv = lax.rsqrt(deg)
    dinv_ref[...] = dinv
    dinv2_ref[...] = 1.0 / deg
    y_ref[...] = dinv * z_ref[...]


def _combine_body(p0_ref, p1_ref, y_ref, dinv_ref, dinv2_ref, c_ref, out_ref):
    s = p0_ref[...] + p1_ref[...] + y_ref[...]
    out_ref[...] = dinv2_ref[...] * s + dinv_ref[...] * c_ref[...]


def _final_body(p0_ref, p1_ref, y_ref, dinv_ref, c_ref, bf_ref, batch_ref,
                out_ref):
    f32 = jnp.float32
    z4 = dinv_ref[...] * (p0_ref[...] + p1_ref[...] + y_ref[...]) + c_ref[...]
    b = jnp.broadcast_to(batch_ref[...], (N_GRAPHS, N_NODES))
    seg = lax.broadcasted_iota(jnp.int32, (N_GRAPHS, N_NODES), 0)
    m = jnp.where(seg == b, 1.0, 0.0).astype(f32)
    sums = jnp.dot(m, z4, preferred_element_type=f32)
    counts = jnp.sum(m, axis=1, keepdims=True)
    out_ref[...] = sums / jnp.maximum(counts, 1.0) + bf_ref[...]


def _deg_kernel_body(dst_hbm, deg_hbm, ones_v, idx_v, zero_v, deg_sp):
    c = lax.axis_index("c")
    s = lax.axis_index("s")
    f32 = jnp.float32

    def fill_zero(i, _):
        zero_v[pl.ds(i * 16, 16)] = jnp.zeros((16,), f32)
        return 0
    lax.fori_loop(0, 40, fill_zero, 0)

    def fill_one(i, _):
        ones_v[pl.ds(i * 16, 16)] = jnp.ones((16,), f32)
        return 0
    lax.fori_loop(0, 8, fill_one, 0)

    pltpu.sync_copy(zero_v, deg_sp.at[pl.ds(s * 640, 640)])
    plsc.subcore_barrier()

    def scatter_ones(k, _):
        b = s + k * 16

        @pl.when(b < N_BATCHES)
        def _():
            pltpu.sync_copy(dst_hbm.at[pl.ds(b * 128, 128)], idx_v.at[0])
            pltpu.sync_copy(ones_v, deg_sp.at[idx_v.at[0]], add=True)
        return 0
    lax.fori_loop(0, (N_BATCHES + 15) // 16, scatter_ones, 0)
    plsc.subcore_barrier()

    @pl.when(c == 0)
    def _():
        pltpu.sync_copy(deg_sp.at[pl.ds(s * 640, 640)],
                        deg_hbm.at[pl.ds(s * 640, 640)])


_deg_kernel = pl.kernel(
    _deg_kernel_body,
    out_type=jax.ShapeDtypeStruct((N_PAD,), jnp.float32),
    mesh=_MESH,
    scratch_types=[
        pltpu.VMEM((128,), jnp.float32),
        pltpu.VMEM((1, 128), jnp.int32),
        pltpu.VMEM((640,), jnp.float32),
        pltpu.VMEM_SHARED((N_PAD,), jnp.float32),
    ],
    compiler_params=_SC_PARAMS,
)


def _scatter_kernel_body(y_hbm, src_hbm, dst_hbm, p_hbm,
                         zbuf, sidx, didx, rows, acc, sem):
    c = lax.axis_index("c")
    s = lax.axis_index("s")
    f32 = jnp.float32

    def fill_zero(i, _):
        zbuf[i, pl.ds(0, 16)] = jnp.zeros((16,), f32)
        zbuf[i, pl.ds(16, 16)] = jnp.zeros((16,), f32)
        return 0
    lax.fori_loop(0, 128, fill_zero, 0)
    for j in range(5):
        pltpu.sync_copy(zbuf, acc.at[pl.ds(s * 640 + j * 128, 128)])
    plsc.subcore_barrier()

    half = N_BATCHES // 2

    def edge_batch(k, _):
        b = c * half + s + k * 16

        @pl.when(b < (c + 1) * half)
        def _():
            pltpu.sync_copy(src_hbm.at[pl.ds(b * 128, 128)], sidx.at[0])
            pltpu.sync_copy(dst_hbm.at[pl.ds(b * 128, 128)], didx.at[0])
            pltpu.async_copy(y_hbm.at[sidx.at[0]], rows, sem).wait()
            pltpu.sync_copy(rows, acc.at[didx.at[0]], add=True)
        return 0
    lax.fori_loop(0, (half + 15) // 16, edge_batch, 0)
    plsc.subcore_barrier()

    pltpu.sync_copy(acc.at[pl.ds(s * 640, 640)],
                    p_hbm.at[c, pl.ds(s * 640, 640)])


_scatter_kernel = pl.kernel(
    _scatter_kernel_body,
    out_type=jax.ShapeDtypeStruct((2, N_PAD, D_OUT), jnp.float32),
    mesh=_MESH,
    scratch_types=[
        pltpu.VMEM((128, D_OUT), jnp.float32),
        pltpu.VMEM((1, 128), jnp.int32),
        pltpu.VMEM((1, 128), jnp.int32),
        pltpu.VMEM((128, D_OUT), jnp.float32),
        pltpu.VMEM_SHARED((N_PAD, D_OUT), jnp.float32),
        pltpu.SemaphoreType.DMA,
    ],
    compiler_params=_SC_PARAMS,
)


def kernel(x, edge_index, batch, W1, b1, W2, b2, W3, b3, W4, b4, Wf, bf):
    f32 = jnp.float32
    src = edge_index[0].astype(jnp.int32)
    dst = edge_index[1].astype(jnp.int32)
    batch32 = batch.astype(jnp.int32).reshape(1, N_NODES)

    z0, cs = pl.pallas_call(
        _prep_body,
        out_shape=[jax.ShapeDtypeStruct((N_NODES, D_OUT), f32),
                   jax.ShapeDtypeStruct((4, D_OUT), f32)],
    )(x, W1, W2, W3, W4, Wf,
      b1.reshape(1, -1), b2.reshape(1, -1), b3.reshape(1, -1),
      b4.reshape(1, -1))

    deg_pad = _deg_kernel(dst)
    y, dinv, dinv2 = pl.pallas_call(
        _scale_body,
        out_shape=[jax.ShapeDtypeStruct((N_NODES, D_OUT), f32),
                   jax.ShapeDtypeStruct((N_NODES, 1), f32),
                   jax.ShapeDtypeStruct((N_NODES, 1), f32)],
    )(z0, deg_pad[:N_NODES].reshape(N_NODES, 1))

    for l in range(3):
        p = _scatter_kernel(y, src, dst)
        y = pl.pallas_call(
            _combine_body,
            out_shape=jax.ShapeDtypeStruct((N_NODES, D_OUT), f32),
        )(p[0, :N_NODES], p[1, :N_NODES], y, dinv, dinv2,
          cs[l:l + 1, :])

    p = _scatter_kernel(y, src, dst)
    out = pl.pallas_call(
        _final_body,
        out_shape=jax.ShapeDtypeStruct((N_GRAPHS, D_OUT), f32),
    )(p[0, :N_NODES], p[1, :N_NODES], y, dinv, cs[3:4, :],
      bf.reshape(1, -1), batch32)
    return out


# speedup vs baseline: 43.0548x; 2.6698x over previous
import functools

import jax
import jax.numpy as jnp
from jax import lax
from jax.experimental import pallas as pl
from jax.experimental.pallas import tpu as pltpu
from jax.experimental.pallas import tpu_sc as plsc

N_NODES = 10000
N_PAD = 10240
N_EDGES = 320000
N_BATCHES = N_EDGES // 128
D_OUT = 32
N_GRAPHS = 64

_MESH = plsc.VectorSubcoreMesh(core_axis_name="c", subcore_axis_name="s")
_SC_PARAMS = pltpu.CompilerParams(use_tc_tiling_on_sc=False)


def _prep_body(x_ref, w1, w2, w3, w4, wf, b1, b2, b3, b4, z0_ref, cs_ref):
    f32 = jnp.float32
    wc4 = jnp.dot(w4[...], wf[...], preferred_element_type=f32)
    wc3 = jnp.dot(w3[...], wc4, preferred_element_type=f32)
    wc2 = jnp.dot(w2[...], wc3, preferred_element_type=f32)
    wc1 = jnp.dot(w1[...], wc2, preferred_element_type=f32)
    z0_ref[...] = jnp.dot(x_ref[...], wc1, preferred_element_type=f32)
    cs_ref[0:1, :] = jnp.dot(b1[...], wc2, preferred_element_type=f32)
    cs_ref[1:2, :] = jnp.dot(b2[...], wc3, preferred_element_type=f32)
    cs_ref[2:3, :] = jnp.dot(b3[...], wc4, preferred_element_type=f32)
    cs_ref[3:4, :] = jnp.dot(b4[...], wf[...], preferred_element_type=f32)


def _scale_body(z_ref, d0_ref, d1_ref, y_ref, dinv_ref, dinv2_ref):
    deg = d0_ref[...] + d1_ref[...] + 1.0
    dinv = lax.rsqrt(deg)
    dinv_ref[...] = dinv
    dinv2_ref[...] = 1.0 / deg
    y_ref[...] = dinv * z_ref[...]


def _combine_body(p0_ref, p1_ref, y_ref, dinv_ref, dinv2_ref, c_ref, out_ref):
    s = p0_ref[...] + p1_ref[...] + y_ref[...]
    out_ref[...] = dinv2_ref[...] * s + dinv_ref[...] * c_ref[...]


def _final_body(p0_ref, p1_ref, y_ref, dinv_ref, c_ref, bf_ref, batch_ref,
                out_ref):
    f32 = jnp.float32
    z4 = dinv_ref[...] * (p0_ref[...] + p1_ref[...] + y_ref[...]) + c_ref[...]
    b = jnp.broadcast_to(batch_ref[...], (N_GRAPHS, N_NODES))
    seg = lax.broadcasted_iota(jnp.int32, (N_GRAPHS, N_NODES), 0)
    m = jnp.where(seg == b, 1.0, 0.0).astype(f32)
    sums = jnp.dot(m, z4, preferred_element_type=f32)
    counts = jnp.sum(m, axis=1, keepdims=True)
    out_ref[...] = sums / jnp.maximum(counts, 1.0) + bf_ref[...]


_HALF = N_BATCHES // 2
_MAXB = 79


def _tile_range(c, s):
    lo = c * _HALF + s * 78 + jnp.minimum(s, 2)
    n = jnp.where(s < 2, 79, 78)
    return lo, n


def _prefetch_idx(idx2d_hbm, lo, nb, buf):
    pltpu.sync_copy(idx2d_hbm.at[pl.ds(lo, 78)], buf.at[pl.ds(0, 78)])

    @pl.when(nb > 78)
    def _():
        pltpu.sync_copy(idx2d_hbm.at[pl.ds(lo + 78, 1)],
                        buf.at[pl.ds(78, 1)])


def _deg_kernel_body(dst2d_hbm, deg_hbm, ones_v, didx, zero_v, sem, deg_sp):
    c = lax.axis_index("c")
    s = lax.axis_index("s")
    f32 = jnp.float32

    def fill_zero(i, _):
        zero_v[pl.ds(i * 16, 16)] = jnp.zeros((16,), f32)
        return 0
    lax.fori_loop(0, 40, fill_zero, 0)

    def fill_one(i, _):
        ones_v[pl.ds(i * 16, 16)] = jnp.ones((16,), f32)
        return 0
    lax.fori_loop(0, 8, fill_one, 0)

    lo, nb = _tile_range(c, s)
    _prefetch_idx(dst2d_hbm, lo, nb, didx)

    pltpu.sync_copy(zero_v, deg_sp.at[pl.ds(s * 640, 640)])
    plsc.subcore_barrier()

    def scatter_ones(b, _):
        @pl.when(b < nb)
        def _():
            pltpu.async_copy(ones_v, deg_sp.at[didx.at[b]], sem, add=True)
        return 0
    lax.fori_loop(0, _MAXB, scatter_ones, 0)

    def drain(b, _):
        @pl.when(b < nb)
        def _():
            pltpu.make_async_copy(ones_v, deg_sp.at[didx.at[b]], sem).wait()
        return 0
    lax.fori_loop(0, _MAXB, drain, 0)
    plsc.subcore_barrier()

    pltpu.sync_copy(deg_sp.at[pl.ds(s * 640, 640)],
                    deg_hbm.at[c, pl.ds(s * 640, 640)])


_deg_kernel = pl.kernel(
    _deg_kernel_body,
    out_type=jax.ShapeDtypeStruct((2, N_PAD), jnp.float32),
    mesh=_MESH,
    scratch_types=[
        pltpu.VMEM((128,), jnp.float32),
        pltpu.VMEM((_MAXB, 128), jnp.int32),
        pltpu.VMEM((640,), jnp.float32),
        pltpu.SemaphoreType.DMA,
        pltpu.VMEM_SHARED((N_PAD,), jnp.float32),
    ],
    compiler_params=_SC_PARAMS,
)


_NBUF = 4


def _scatter_kernel_body(y_hbm, src2d_hbm, dst2d_hbm, p_hbm,
                         zbuf, sidx, didx,
                         rows0, rows1, rows2, rows3,
                         gs0, gs1, gs2, gs3, ss0, ss1, ss2, ss3,
                         acc):
    c = lax.axis_index("c")
    s = lax.axis_index("s")
    f32 = jnp.float32
    rows = [rows0, rows1, rows2, rows3]
    gsem = [gs0, gs1, gs2, gs3]
    ssem = [ss0, ss1, ss2, ss3]

    def fill_zero(i, _):
        zbuf[i, pl.ds(0, 16)] = jnp.zeros((16,), f32)
        zbuf[i, pl.ds(16, 16)] = jnp.zeros((16,), f32)
        return 0
    lax.fori_loop(0, 128, fill_zero, 0)

    lo, nb = _tile_range(c, s)
    _prefetch_idx(src2d_hbm, lo, nb, sidx)
    _prefetch_idx(dst2d_hbm, lo, nb, didx)

    for j in range(5):
        pltpu.sync_copy(zbuf, acc.at[pl.ds(s * 640 + j * 128, 128)])
    plsc.subcore_barrier()

    def group(g, _):
        for j in range(_NBUF):
            b = g * _NBUF + j

            @pl.when(b < nb)
            def _():
                @pl.when(g > 0)
                def _():
                    pltpu.make_async_copy(rows[j], acc.at[didx.at[b]],
                                          ssem[j]).wait()
                pltpu.async_copy(y_hbm.at[sidx.at[b]], rows[j], gsem[j])
        for j in range(_NBUF):
            b = g * _NBUF + j

            @pl.when(b < nb)
            def _():
                pltpu.make_async_copy(y_hbm.at[sidx.at[b]], rows[j],
                                      gsem[j]).wait()
                pltpu.async_copy(rows[j], acc.at[didx.at[b]], ssem[j],
                                 add=True)
        return 0
    lax.fori_loop(0, (_MAXB + _NBUF - 1) // _NBUF, group, 0)

    for j in range(_NBUF):
        pltpu.make_async_copy(rows[j], acc.at[didx.at[0]], ssem[j]).wait()
    plsc.subcore_barrier()

    pltpu.sync_copy(acc.at[pl.ds(s * 640, 640)],
                    p_hbm.at[c, pl.ds(s * 640, 640)])


_scatter_kernel = pl.kernel(
    _scatter_kernel_body,
    out_type=jax.ShapeDtypeStruct((2, N_PAD, D_OUT), jnp.float32),
    mesh=_MESH,
    scratch_types=(
        [pltpu.VMEM((128, D_OUT), jnp.float32),
         pltpu.VMEM((_MAXB, 128), jnp.int32),
         pltpu.VMEM((_MAXB, 128), jnp.int32)]
        + [pltpu.VMEM((128, D_OUT), jnp.float32) for _ in range(_NBUF)]
        + [pltpu.SemaphoreType.DMA for _ in range(2 * _NBUF)]
        + [pltpu.VMEM_SHARED((N_PAD, D_OUT), jnp.float32)]
    ),
    compiler_params=_SC_PARAMS,
)


def kernel(x, edge_index, batch, W1, b1, W2, b2, W3, b3, W4, b4, Wf, bf):
    f32 = jnp.float32
    src = edge_index[0].astype(jnp.int32)
    dst = edge_index[1].astype(jnp.int32)
    batch32 = batch.astype(jnp.int32).reshape(1, N_NODES)

    z0, cs = pl.pallas_call(
        _prep_body,
        out_shape=[jax.ShapeDtypeStruct((N_NODES, D_OUT), f32),
                   jax.ShapeDtypeStruct((4, D_OUT), f32)],
    )(x, W1, W2, W3, W4, Wf,
      b1.reshape(1, -1), b2.reshape(1, -1), b3.reshape(1, -1),
      b4.reshape(1, -1))

    src2d = src.reshape(N_BATCHES, 128)
    dst2d = dst.reshape(N_BATCHES, 128)

    deg_pad = _deg_kernel(dst2d)
    y, dinv, dinv2 = pl.pallas_call(
        _scale_body,
        out_shape=[jax.ShapeDtypeStruct((N_NODES, D_OUT), f32),
                   jax.ShapeDtypeStruct((N_NODES, 1), f32),
                   jax.ShapeDtypeStruct((N_NODES, 1), f32)],
    )(z0, deg_pad[0, :N_NODES].reshape(N_NODES, 1),
      deg_pad[1, :N_NODES].reshape(N_NODES, 1))

    for l in range(3):
        p = _scatter_kernel(y, src2d, dst2d)
        y = pl.pallas_call(
            _combine_body,
            out_shape=jax.ShapeDtypeStruct((N_NODES, D_OUT), f32),
        )(p[0, :N_NODES], p[1, :N_NODES], y, dinv, dinv2,
          cs[l:l + 1, :])

    p = _scatter_kernel(y, src2d, dst2d)
    out = pl.pallas_call(
        _final_body,
        out_shape=jax.ShapeDtypeStruct((N_GRAPHS, D_OUT), f32),
    )(p[0, :N_NODES], p[1, :N_NODES], y, dinv, cs[3:4, :],
      bf.reshape(1, -1), batch32)
    return out


# speedup vs baseline: 47.3456x; 1.0997x over previous
import functools

import jax
import jax.numpy as jnp
from jax import lax
from jax.experimental import pallas as pl
from jax.experimental.pallas import tpu as pltpu
from jax.experimental.pallas import tpu_sc as plsc

N_NODES = 10000
N_PAD = 10240
N_EDGES = 320000
N_BATCHES = N_EDGES // 128
D_OUT = 32
N_GRAPHS = 64

_MESH = plsc.VectorSubcoreMesh(core_axis_name="c", subcore_axis_name="s")
_SC_PARAMS = pltpu.CompilerParams(use_tc_tiling_on_sc=False)


def _prep_body(x_ref, w1, w2, w3, w4, wf, b1, b2, b3, b4, z0_ref, cs_ref):
    f32 = jnp.float32
    wc4 = jnp.dot(w4[...], wf[...], preferred_element_type=f32)
    wc3 = jnp.dot(w3[...], wc4, preferred_element_type=f32)
    wc2 = jnp.dot(w2[...], wc3, preferred_element_type=f32)
    wc1 = jnp.dot(w1[...], wc2, preferred_element_type=f32)
    z0_ref[0:N_NODES, :] = jnp.dot(x_ref[...], wc1, preferred_element_type=f32)
    z0_ref[N_NODES:N_PAD, :] = jnp.zeros((N_PAD - N_NODES, D_OUT), f32)
    cs_ref[0:1, :] = jnp.dot(b1[...], wc2, preferred_element_type=f32)
    cs_ref[1:2, :] = jnp.dot(b2[...], wc3, preferred_element_type=f32)
    cs_ref[2:3, :] = jnp.dot(b3[...], wc4, preferred_element_type=f32)
    cs_ref[3:4, :] = jnp.dot(b4[...], wf[...], preferred_element_type=f32)


def _scale_body(z_ref, degt_ref, y_ref, dinv_ref, dinv2_ref):
    deg = degt_ref[:, 0:1] + degt_ref[:, 1:2] + 1.0
    dinv = lax.rsqrt(deg)
    dinv_ref[...] = dinv
    dinv2_ref[...] = 1.0 / deg
    y_ref[...] = dinv * z_ref[...]


def _combine_body(p_ref, y_ref, dinv_ref, dinv2_ref, cs_ref, out_ref, *, l):
    s = p_ref[0] + p_ref[1] + y_ref[...]
    out_ref[...] = dinv2_ref[...] * s + dinv_ref[...] * cs_ref[l:l + 1, :]


def _final_body(p_ref, y_ref, dinv_ref, cs_ref, bf_ref, batch_ref, out_ref):
    f32 = jnp.float32
    z4 = (dinv_ref[...] * (p_ref[0] + p_ref[1] + y_ref[...])
          + cs_ref[3:4, :])
    b = jnp.broadcast_to(batch_ref[...], (N_GRAPHS, N_NODES))
    seg = lax.broadcasted_iota(jnp.int32, (N_GRAPHS, N_NODES), 0)
    m = jnp.where(seg == b, 1.0, 0.0).astype(f32)
    sums = jnp.dot(m, z4[0:N_NODES, :], preferred_element_type=f32)
    counts = jnp.sum(m, axis=1, keepdims=True)
    out_ref[...] = sums / jnp.maximum(counts, 1.0) + bf_ref[...]


_HALF = N_BATCHES // 2
_MAXB = 79


def _tile_range(c, s):
    lo = c * _HALF + s * 78 + jnp.minimum(s, 2)
    n = jnp.where(s < 2, 79, 78)
    return lo, n


def _prefetch_idx(idx2d_hbm, lo, nb, buf):
    pltpu.sync_copy(idx2d_hbm.at[pl.ds(lo, 78)], buf.at[pl.ds(0, 78)])

    @pl.when(nb > 78)
    def _():
        pltpu.sync_copy(idx2d_hbm.at[pl.ds(lo + 78, 1)],
                        buf.at[pl.ds(78, 1)])


def _deg_kernel_body(dst2d_hbm, deg_hbm, ones_v, didx, zero_v, sem, deg_sp):
    c = lax.axis_index("c")
    s = lax.axis_index("s")
    f32 = jnp.float32

    def fill_zero(i, _):
        zero_v[pl.ds(i * 16, 16)] = jnp.zeros((16,), f32)
        return 0
    lax.fori_loop(0, 40, fill_zero, 0)

    def fill_one(i, _):
        ones_v[pl.ds(i * 16, 16)] = jnp.ones((16,), f32)
        return 0
    lax.fori_loop(0, 8, fill_one, 0)

    lo, nb = _tile_range(c, s)
    _prefetch_idx(dst2d_hbm, lo, nb, didx)

    pltpu.sync_copy(zero_v, deg_sp.at[pl.ds(s * 640, 640)])
    plsc.subcore_barrier()

    def scatter_ones(b, _):
        @pl.when(b < nb)
        def _():
            pltpu.async_copy(ones_v, deg_sp.at[didx.at[b]], sem, add=True)
        return 0
    lax.fori_loop(0, _MAXB, scatter_ones, 0)

    def drain(b, _):
        @pl.when(b < nb)
        def _():
            pltpu.make_async_copy(ones_v, deg_sp.at[didx.at[b]], sem).wait()
        return 0
    lax.fori_loop(0, _MAXB, drain, 0)
    plsc.subcore_barrier()

    pltpu.sync_copy(deg_sp.at[pl.ds(s * 640, 640)],
                    deg_hbm.at[c, pl.ds(s * 640, 640)])


_deg_kernel = pl.kernel(
    _deg_kernel_body,
    out_type=jax.ShapeDtypeStruct((2, N_PAD), jnp.float32),
    mesh=_MESH,
    scratch_types=[
        pltpu.VMEM((128,), jnp.float32),
        pltpu.VMEM((_MAXB, 128), jnp.int32),
        pltpu.VMEM((640,), jnp.float32),
        pltpu.SemaphoreType.DMA,
        pltpu.VMEM_SHARED((N_PAD,), jnp.float32),
    ],
    compiler_params=_SC_PARAMS,
)


_NBUF = 4


def _scatter_kernel_body(y_hbm, src2d_hbm, dst2d_hbm, p_hbm,
                         zbuf, sidx, didx,
                         rows0, rows1, rows2, rows3,
                         gs0, gs1, gs2, gs3, ss0, ss1, ss2, ss3,
                         acc):
    c = lax.axis_index("c")
    s = lax.axis_index("s")
    f32 = jnp.float32
    rows = [rows0, rows1, rows2, rows3]
    gsem = [gs0, gs1, gs2, gs3]
    ssem = [ss0, ss1, ss2, ss3]

    def fill_zero(i, _):
        zbuf[i, pl.ds(0, 16)] = jnp.zeros((16,), f32)
        zbuf[i, pl.ds(16, 16)] = jnp.zeros((16,), f32)
        return 0
    lax.fori_loop(0, 128, fill_zero, 0)

    lo, nb = _tile_range(c, s)
    _prefetch_idx(src2d_hbm, lo, nb, sidx)
    _prefetch_idx(dst2d_hbm, lo, nb, didx)

    for j in range(5):
        pltpu.sync_copy(zbuf, acc.at[pl.ds(s * 640 + j * 128, 128)])
    plsc.subcore_barrier()

    def group(g, _):
        for j in range(_NBUF):
            b = g * _NBUF + j

            @pl.when(b < nb)
            def _():
                @pl.when(g > 0)
                def _():
                    pltpu.make_async_copy(rows[j], acc.at[didx.at[b]],
                                          ssem[j]).wait()
                pltpu.async_copy(y_hbm.at[sidx.at[b]], rows[j], gsem[j])
        for j in range(_NBUF):
            b = g * _NBUF + j

            @pl.when(b < nb)
            def _():
                pltpu.make_async_copy(y_hbm.at[sidx.at[b]], rows[j],
                                      gsem[j]).wait()
                pltpu.async_copy(rows[j], acc.at[didx.at[b]], ssem[j],
                                 add=True)
        return 0
    lax.fori_loop(0, (_MAXB + _NBUF - 1) // _NBUF, group, 0)

    for j in range(_NBUF):
        pltpu.make_async_copy(rows[j], acc.at[didx.at[0]], ssem[j]).wait()
    plsc.subcore_barrier()

    pltpu.sync_copy(acc.at[pl.ds(s * 640, 640)],
                    p_hbm.at[c, pl.ds(s * 640, 640)])


_scatter_kernel = pl.kernel(
    _scatter_kernel_body,
    out_type=jax.ShapeDtypeStruct((2, N_PAD, D_OUT), jnp.float32),
    mesh=_MESH,
    scratch_types=(
        [pltpu.VMEM((128, D_OUT), jnp.float32),
         pltpu.VMEM((_MAXB, 128), jnp.int32),
         pltpu.VMEM((_MAXB, 128), jnp.int32)]
        + [pltpu.VMEM((128, D_OUT), jnp.float32) for _ in range(_NBUF)]
        + [pltpu.SemaphoreType.DMA for _ in range(2 * _NBUF)]
        + [pltpu.VMEM_SHARED((N_PAD, D_OUT), jnp.float32)]
    ),
    compiler_params=_SC_PARAMS,
)


def kernel(x, edge_index, batch, W1, b1, W2, b2, W3, b3, W4, b4, Wf, bf):
    f32 = jnp.float32
    src = edge_index[0].astype(jnp.int32)
    dst = edge_index[1].astype(jnp.int32)
    batch32 = batch.astype(jnp.int32).reshape(1, N_NODES)

    z0, cs = pl.pallas_call(
        _prep_body,
        out_shape=[jax.ShapeDtypeStruct((N_PAD, D_OUT), f32),
                   jax.ShapeDtypeStruct((4, D_OUT), f32)],
    )(x, W1, W2, W3, W4, Wf,
      b1.reshape(1, -1), b2.reshape(1, -1), b3.reshape(1, -1),
      b4.reshape(1, -1))

    src2d = src.reshape(N_BATCHES, 128)
    dst2d = dst.reshape(N_BATCHES, 128)

    deg_pad = _deg_kernel(dst2d)
    y, dinv, dinv2 = pl.pallas_call(
        _scale_body,
        out_shape=[jax.ShapeDtypeStruct((N_PAD, D_OUT), f32),
                   jax.ShapeDtypeStruct((N_PAD, 1), f32),
                   jax.ShapeDtypeStruct((N_PAD, 1), f32)],
    )(z0, deg_pad.T)

    for l in range(3):
        p = _scatter_kernel(y, src2d, dst2d)
        y = pl.pallas_call(
            functools.partial(_combine_body, l=l),
            out_shape=jax.ShapeDtypeStruct((N_PAD, D_OUT), f32),
        )(p, y, dinv, dinv2, cs)

    p = _scatter_kernel(y, src2d, dst2d)
    out = pl.pallas_call(
        _final_body,
        out_shape=jax.ShapeDtypeStruct((N_GRAPHS, D_OUT), f32),
    )(p, y, dinv, cs, bf.reshape(1, -1), batch32)
    return out


# speedup vs baseline: 65.2699x; 1.3786x over previous
import functools

import jax
import jax.numpy as jnp
from jax import lax
from jax.experimental import pallas as pl
from jax.experimental.pallas import tpu as pltpu
from jax.experimental.pallas import tpu_sc as plsc

N_NODES = 10000
N_PAD = 10240
N_EDGES = 320000
N_BATCHES = N_EDGES // 128
D_OUT = 32
N_GRAPHS = 64

_MESH = plsc.VectorSubcoreMesh(core_axis_name="c", subcore_axis_name="s")
_SC_PARAMS = pltpu.CompilerParams(use_tc_tiling_on_sc=False)


def _prep_body(x_ref, w1, w2, w3, w4, wf, b1, b2, b3, b4, z0_ref, cs_ref):
    f32 = jnp.float32
    wc4 = jnp.dot(w4[...], wf[...], preferred_element_type=f32)
    wc3 = jnp.dot(w3[...], wc4, preferred_element_type=f32)
    wc2 = jnp.dot(w2[...], wc3, preferred_element_type=f32)
    wc1 = jnp.dot(w1[...], wc2, preferred_element_type=f32)
    z0_ref[0:N_NODES, :] = jnp.dot(x_ref[...], wc1, preferred_element_type=f32)
    z0_ref[N_NODES:N_PAD, :] = jnp.zeros((N_PAD - N_NODES, D_OUT), f32)
    c1 = jnp.dot(b1[...], wc2, preferred_element_type=f32)
    c2 = jnp.dot(b2[...], wc3, preferred_element_type=f32)
    c3 = jnp.dot(b3[...], wc4, preferred_element_type=f32)
    c4 = jnp.dot(b4[...], wf[...], preferred_element_type=f32)
    cs_ref[0:1, :] = jnp.concatenate([c1, c1, c1, c1], axis=1)
    cs_ref[1:2, :] = jnp.concatenate([c2, c2, c2, c2], axis=1)
    cs_ref[2:3, :] = jnp.concatenate([c3, c3, c3, c3], axis=1)
    cs_ref[3:4, :] = jnp.concatenate([c4, c4, c4, c4], axis=1)


_N_E = N_PAD // 4


def _expand4(v4):
    a = lax.broadcasted_iota(jnp.int32, (4, 128), 0)
    l = lax.broadcasted_iota(jnp.int32, (4, 128), 1) // D_OUT
    e = jnp.where(a == l, 1.0, 0.0).astype(jnp.float32)
    return jnp.dot(v4, e, preferred_element_type=jnp.float32)


def _scale_body(z_ref, d0_ref, d1_ref, y_ref, dinv_ref, dinv2_ref,
                dinvc_ref):
    deg4 = d0_ref[...] + d1_ref[...] + 1.0
    dinv4 = lax.rsqrt(deg4)
    dinvE = _expand4(dinv4)
    dinv_ref[...] = dinvE
    dinv2_ref[...] = _expand4(1.0 / deg4)
    y_ref[...] = dinvE * z_ref[...]
    dinvc_ref[...] = dinv4


def _combine_body(p_ref, y_ref, dinv_ref, dinv2_ref, cs_ref, out_ref, *, l):
    s = p_ref[0] + p_ref[1] + y_ref[...]
    out_ref[...] = dinv2_ref[...] * s + dinv_ref[...] * cs_ref[l:l + 1, :]


def _final_body(p_ref, y_ref, dinv_ref, cs_ref, bf_ref, batch_ref, out_ref):
    f32 = jnp.float32
    z4 = (dinv_ref[...] * (p_ref[0] + p_ref[1] + y_ref[...])
          + cs_ref[3:4, 0:D_OUT])
    b = jnp.broadcast_to(batch_ref[...], (N_GRAPHS, N_NODES))
    seg = lax.broadcasted_iota(jnp.int32, (N_GRAPHS, N_NODES), 0)
    m = jnp.where(seg == b, 1.0, 0.0).astype(f32)
    sums = jnp.dot(m, z4[0:N_NODES, :], preferred_element_type=f32)
    counts = jnp.sum(m, axis=1, keepdims=True)
    out_ref[...] = sums / jnp.maximum(counts, 1.0) + bf_ref[...]


_HALF = N_BATCHES // 2
_MAXB = 79


def _tile_range(c, s):
    lo = c * _HALF + s * 78 + jnp.minimum(s, 2)
    n = jnp.where(s < 2, 79, 78)
    return lo, n


def _prefetch_idx(idx2d_hbm, lo, nb, buf):
    pltpu.sync_copy(idx2d_hbm.at[pl.ds(lo, 78)], buf.at[pl.ds(0, 78)])

    @pl.when(nb > 78)
    def _():
        pltpu.sync_copy(idx2d_hbm.at[pl.ds(lo + 78, 1)],
                        buf.at[pl.ds(78, 1)])


def _deg_kernel_body(dst2d_hbm, deg_hbm, ones_v, didx, zero_v, sem, deg_sp):
    c = lax.axis_index("c")
    s = lax.axis_index("s")
    f32 = jnp.float32

    def fill_zero(i, _):
        zero_v[pl.ds(i * 16, 16)] = jnp.zeros((16,), f32)
        return 0
    lax.fori_loop(0, 40, fill_zero, 0)

    def fill_one(i, _):
        ones_v[pl.ds(i * 16, 16)] = jnp.ones((16,), f32)
        return 0
    lax.fori_loop(0, 8, fill_one, 0)

    lo, nb = _tile_range(c, s)
    _prefetch_idx(dst2d_hbm, lo, nb, didx)

    pltpu.sync_copy(zero_v, deg_sp.at[pl.ds(s * 640, 640)])
    plsc.subcore_barrier()

    def scatter_ones(b, _):
        @pl.when(b < nb)
        def _():
            pltpu.async_copy(ones_v, deg_sp.at[didx.at[b]], sem, add=True)
        return 0
    lax.fori_loop(0, _MAXB, scatter_ones, 0)

    def drain(b, _):
        @pl.when(b < nb)
        def _():
            pltpu.make_async_copy(ones_v, deg_sp.at[didx.at[b]], sem).wait()
        return 0
    lax.fori_loop(0, _MAXB, drain, 0)
    plsc.subcore_barrier()

    pltpu.sync_copy(deg_sp.at[pl.ds(s * 640, 640)],
                    deg_hbm.at[c, pl.ds(s * 640, 640)])


_deg_kernel = pl.kernel(
    _deg_kernel_body,
    out_type=jax.ShapeDtypeStruct((2, N_PAD), jnp.float32),
    mesh=_MESH,
    scratch_types=[
        pltpu.VMEM((128,), jnp.float32),
        pltpu.VMEM((_MAXB, 128), jnp.int32),
        pltpu.VMEM((640,), jnp.float32),
        pltpu.SemaphoreType.DMA,
        pltpu.VMEM_SHARED((N_PAD,), jnp.float32),
    ],
    compiler_params=_SC_PARAMS,
)


_NBUF = 8


def _scatter_kernel_body(y_hbm, src2d_hbm, dst2d_hbm, p_hbm,
                         zbuf, sidx, didx, *bufs_and_sems):
    c = lax.axis_index("c")
    s = lax.axis_index("s")
    f32 = jnp.float32
    rows = list(bufs_and_sems[:_NBUF])
    gsem = list(bufs_and_sems[_NBUF:2 * _NBUF])
    ssem = list(bufs_and_sems[2 * _NBUF:3 * _NBUF])
    acc = bufs_and_sems[3 * _NBUF]

    def fill_zero(i, _):
        zbuf[i, pl.ds(0, 16)] = jnp.zeros((16,), f32)
        zbuf[i, pl.ds(16, 16)] = jnp.zeros((16,), f32)
        return 0
    lax.fori_loop(0, 128, fill_zero, 0)

    lo, nb = _tile_range(c, s)
    _prefetch_idx(src2d_hbm, lo, nb, sidx)
    _prefetch_idx(dst2d_hbm, lo, nb, didx)

    for j in range(5):
        pltpu.sync_copy(zbuf, acc.at[pl.ds(s * 640 + j * 128, 128)])
    plsc.subcore_barrier()

    def group(g, _):
        for j in range(_NBUF):
            b = g * _NBUF + j

            @pl.when(b < nb)
            def _():
                @pl.when(g > 0)
                def _():
                    pltpu.make_async_copy(rows[j], acc.at[didx.at[b]],
                                          ssem[j]).wait()
                pltpu.async_copy(y_hbm.at[sidx.at[b]], rows[j], gsem[j])
        for j in range(_NBUF):
            b = g * _NBUF + j

            @pl.when(b < nb)
            def _():
                pltpu.make_async_copy(y_hbm.at[sidx.at[b]], rows[j],
                                      gsem[j]).wait()
                pltpu.async_copy(rows[j], acc.at[didx.at[b]], ssem[j],
                                 add=True)
        return 0
    lax.fori_loop(0, (_MAXB + _NBUF - 1) // _NBUF, group, 0)

    for j in range(_NBUF):
        pltpu.make_async_copy(rows[j], acc.at[didx.at[0]], ssem[j]).wait()
    plsc.subcore_barrier()

    pltpu.sync_copy(acc.at[pl.ds(s * 640, 640)],
                    p_hbm.at[c, pl.ds(s * 640, 640)])


_scatter_kernel = pl.kernel(
    _scatter_kernel_body,
    out_type=jax.ShapeDtypeStruct((2, N_PAD, D_OUT), jnp.float32),
    mesh=_MESH,
    scratch_types=(
        [pltpu.VMEM((128, D_OUT), jnp.float32),
         pltpu.VMEM((_MAXB, 128), jnp.int32),
         pltpu.VMEM((_MAXB, 128), jnp.int32)]
        + [pltpu.VMEM((128, D_OUT), jnp.float32) for _ in range(_NBUF)]
        + [pltpu.SemaphoreType.DMA for _ in range(2 * _NBUF)]
        + [pltpu.VMEM_SHARED((N_PAD, D_OUT), jnp.float32)]
    ),
    compiler_params=_SC_PARAMS,
)


def kernel(x, edge_index, batch, W1, b1, W2, b2, W3, b3, W4, b4, Wf, bf):
    f32 = jnp.float32
    src = edge_index[0].astype(jnp.int32)
    dst = edge_index[1].astype(jnp.int32)
    batch32 = batch.astype(jnp.int32).reshape(1, N_NODES)

    z0, cs4 = pl.pallas_call(
        _prep_body,
        out_shape=[jax.ShapeDtypeStruct((N_PAD, D_OUT), f32),
                   jax.ShapeDtypeStruct((4, 128), f32)],
    )(x, W1, W2, W3, W4, Wf,
      b1.reshape(1, -1), b2.reshape(1, -1), b3.reshape(1, -1),
      b4.reshape(1, -1))

    src2d = src.reshape(N_BATCHES, 128)
    dst2d = dst.reshape(N_BATCHES, 128)

    deg_pad = _deg_kernel(dst2d)
    ye, dinv_e, dinv2_e, dinv4 = pl.pallas_call(
        _scale_body,
        out_shape=[jax.ShapeDtypeStruct((_N_E, 128), f32),
                   jax.ShapeDtypeStruct((_N_E, 128), f32),
                   jax.ShapeDtypeStruct((_N_E, 128), f32),
                   jax.ShapeDtypeStruct((_N_E, 4), f32)],
    )(z0.reshape(_N_E, 128),
      deg_pad[0].reshape(_N_E, 4), deg_pad[1].reshape(_N_E, 4))
    dinv = dinv4.reshape(N_PAD, 1)

    for l in range(3):
        p = _scatter_kernel(ye.reshape(N_PAD, D_OUT), src2d, dst2d)
        ye = pl.pallas_call(
            functools.partial(_combine_body, l=l),
            out_shape=jax.ShapeDtypeStruct((_N_E, 128), f32),
        )(p.reshape(2, _N_E, 128), ye, dinv_e, dinv2_e, cs4)

    p = _scatter_kernel(ye.reshape(N_PAD, D_OUT), src2d, dst2d)
    out = pl.pallas_call(
        _final_body,
        out_shape=jax.ShapeDtypeStruct((N_GRAPHS, D_OUT), f32),
    )(p, ye.reshape(N_PAD, D_OUT), dinv, cs4, bf.reshape(1, -1), batch32)
    return out
